# Initial kernel scaffold; baseline (speedup 1.0000x reference)
import jax
import jax.numpy as jnp
from jax.experimental import pallas as pl


def kernel(id, value, emb_table, lin_weight, lin_bias):
    raise NotImplementedError("write your pallas kernel here")


# speedup vs baseline: 1.2194x; 1.2194x over previous
import functools
import jax
import jax.numpy as jnp
from jax import lax
from jax.experimental import pallas as pl
from jax.experimental.pallas import tpu as pltpu
from jax.experimental.pallas import tpu_sc as plsc

B, F, NEMB = 16384, 26, 16
NW = 32
RPW = B // NW
CHR = 64
NCH = RPW // CHR
IPC = CHR * F
GSZ = 128
GPC = IPC // GSZ
FP = 32


def _bcast_lane(v, j):
    idx = jnp.full((16, 1), j, jnp.int32)
    dnums = lax.GatherDimensionNumbers(
        offset_dims=(), collapsed_slice_dims=(0,), start_index_map=(0,))
    return lax.gather(v, idx, dnums, (1,),
                      mode=lax.GatherScatterMode.PROMISE_IN_BOUNDS)


def _worker_body(ids_hbm, val_hbm, emb_hbm, lw_hbm, out_hbm,
                 idx_v, val_v, ebuf, wbuf, obuf, sem):
    nc = 2
    wid = lax.axis_index("s") * nc + lax.axis_index("c")
    base = wid * RPW

    pltpu.sync_copy(ids_hbm.at[wid], idx_v)
    pltpu.sync_copy(val_hbm.at[pl.ds(base, RPW)], val_v)

    def chunk_body(c, _):
        waits = []
        for j in range(GPC):
            isl = idx_v.at[c * GPC + j]
            waits.append(pltpu.async_copy(
                emb_hbm.at[isl], ebuf.at[pl.ds(j * GSZ, GSZ)], sem))
            waits.append(pltpu.async_copy(
                lw_hbm.at[isl], wbuf.at[pl.ds(j * GSZ, GSZ)], sem))
        for w in waits:
            w.wait()

        lanes = lax.iota(jnp.int32, 16)

        def row_body(r, ov):
            rr = c * CHR + r
            vv0 = val_v[rr, pl.ds(0, 16)]
            vv1 = val_v[rr, pl.ds(16, 16)]
            acc = jnp.zeros((16,), jnp.float32)
            acc2 = jnp.zeros((16,), jnp.float32)
            for f in range(F):
                vv = vv0 if f < 16 else vv1
                sv = _bcast_lane(vv, f % 16)
                e = ebuf[r * F + f]
                t = e * sv
                acc = acc + t
                acc2 = acc2 + t * t
            w0 = wbuf[pl.ds(r * F, 16)]
            w1 = wbuf[pl.ds(r * F + 16, 16)]
            z = 0.5 * (acc * acc - acc2) + w0 * vv0 + w1 * vv1
            ov = jnp.where(lanes == (r & 15), jnp.sum(z), ov)

            @pl.when((r & 15) == 15)
            def _():
                obuf[rr >> 4] = ov

            return ov

        lax.fori_loop(0, CHR, row_body, jnp.zeros((16,), jnp.float32),
                      unroll=False)
        return 0

    lax.fori_loop(0, NCH, chunk_body, 0, unroll=False)
    pltpu.sync_copy(obuf, out_hbm.at[pl.ds(wid * (RPW // 16), RPW // 16)])


@jax.jit
def _fm_sc(ids3, val_p, emb_table, lin_weight):
    mesh = plsc.VectorSubcoreMesh(core_axis_name="c", subcore_axis_name="s")
    grid_kernel = pl.kernel(
        _worker_body,
        out_type=jax.ShapeDtypeStruct((B // 16, 16), jnp.float32),
        mesh=mesh,
        compiler_params=pltpu.CompilerParams(
            needs_layout_passes=False, use_tc_tiling_on_sc=False),
        scratch_types=[
            pltpu.VMEM((IPC * NCH // GSZ, GSZ), jnp.int32),
            pltpu.VMEM((RPW, FP), jnp.float32),
            pltpu.VMEM((IPC, NEMB), jnp.float32),
            pltpu.VMEM((IPC + 16,), jnp.float32),
            pltpu.VMEM((RPW // 16, 16), jnp.float32),
            pltpu.SemaphoreType.DMA,
        ],
    )
    return grid_kernel(ids3, val_p, emb_table, lin_weight)


def kernel(id, value, emb_table, lin_weight, lin_bias):
    ids3 = id.astype(jnp.int32).reshape(NW, RPW * F // GSZ, GSZ)
    val_p = jnp.pad(value, ((0, 0), (0, FP - F)))
    out = _fm_sc(ids3, val_p, emb_table, lin_weight)
    return out.reshape(B) + lin_bias


# speedup vs baseline: 1.9520x; 1.6008x over previous
import jax
import jax.numpy as jnp
from jax import lax
from jax.experimental import pallas as pl
from jax.experimental.pallas import tpu as pltpu
from jax.experimental.pallas import tpu_sc as plsc

B, F, NFEAT, NEMB = 16384, 26, 1000000, 16
NW = 32
RPW = B // NW
CHR = 64
NCH = RPW // CHR
NG = CHR // 16

NBLK = NFEAT // 128
BPW = NBLK // NW
NREST = NBLK - BPW * NW


def _bcast_lane(v, j):
    idx = jnp.full((16, 1), j, jnp.int32)
    dnums = lax.GatherDimensionNumbers(
        offset_dims=(), collapsed_slice_dims=(0,), start_index_map=(0,))
    return lax.gather(v, idx, dnums, (1,),
                      mode=lax.GatherScatterMode.PROMISE_IN_BOUNDS)


def _lanes():
    return lax.iota(jnp.int32, 16)


def _idiv8():
    return lax.iota(jnp.int32, 16) // 8


def _imod8():
    return lax.iota(jnp.int32, 16) % 8


def _transpose_block(tbuf, obuf_v, rows):
    ln = _lanes()
    for rp in range(rows):
        segs = [
            plsc.load_gather(
                tbuf, [ln, jnp.full((16,), 8 * rp + s, jnp.int32)])
            for s in range(8)
        ]
        for s in range(8):
            obuf_v[rp, pl.ds(16 * s, 16)] = segs[s]


SB = 8
NSUP = BPW // SB
STAIL = BPW - NSUP * SB


def _transpose_dyn(tbuf, obuf_v, blk):
    ln = _lanes()
    cb = jnp.full((16,), blk * 128, jnp.int32)
    for rp in range(16):
        segs = [
            plsc.load_gather(
                tbuf, [ln, cb + jnp.full((16,), 8 * rp + s, jnp.int32)])
            for s in range(8)
        ]
        for s in range(8):
            obuf_v[blk * 16 + rp, pl.ds(16 * s, 16)] = segs[s]


def _relayout_body(embT_hbm, out_hbm, *bufs):
    nc = 2
    wid = lax.axis_index("s") * nc + lax.axis_index("c")
    tbs, obs = bufs[0:2], bufs[2:4]
    sis, sos = bufs[4:6], bufs[6:8]

    def in_copy(j, par, start):
        b = wid * BPW + j * SB
        for h in range(2):
            cp = pltpu.make_async_copy(
                embT_hbm.at[pl.ds(8 * h, 8), pl.ds(b * 128, SB * 128)],
                tbs[par].at[pl.ds(8 * h, 8)], sis[par])
            cp.start() if start else cp.wait()

    def out_copy(j, par, start):
        b = wid * BPW + j * SB
        cp = pltpu.make_async_copy(
            obs[par], out_hbm.at[pl.ds(b * 16, SB * 16)], sos[par])
        cp.start() if start else cp.wait()

    in_copy(0, 0, True)

    def jj_body(jj, _):
        for par in range(2):
            j = jj * 2 + par

            @pl.when(j + 1 < NSUP)
            def _():
                in_copy(j + 1, 1 - par, True)

            in_copy(j, par, False)

            @pl.when(j >= 2)
            def _():
                out_copy(j - 2, par, False)

            lax.fori_loop(
                0, SB,
                lambda blk, c: (_transpose_dyn(tbs[par], obs[par], blk), c)[1],
                0, unroll=False)
            out_copy(j, par, True)
        return 0

    lax.fori_loop(0, NSUP // 2, jj_body, 0, unroll=False)
    out_copy(NSUP - 2, 0, False)
    out_copy(NSUP - 1, 1, False)

    bt = wid * BPW + NSUP * SB
    for h in range(2):
        pltpu.sync_copy(
            embT_hbm.at[pl.ds(8 * h, 8), pl.ds(bt * 128, STAIL * 128)],
            tbs[0].at[pl.ds(8 * h, 8), pl.ds(0, STAIL * 128)])
    lax.fori_loop(
        0, STAIL,
        lambda blk, c: (_transpose_dyn(tbs[0], obs[0], blk), c)[1],
        0, unroll=False)
    pltpu.sync_copy(obs[0].at[pl.ds(0, STAIL * 16)],
                    out_hbm.at[pl.ds(bt * 16, STAIL * 16)])

    for w in range(NREST):
        @pl.when(wid == w)
        def _():
            b = NBLK - NREST + w
            for h in range(2):
                pltpu.sync_copy(
                    embT_hbm.at[pl.ds(8 * h, 8), pl.ds(b * 128, 128)],
                    tbs[0].at[pl.ds(8 * h, 8), pl.ds(0, 128)])
            _transpose_dyn(tbs[0], obs[0], 0)
            pltpu.sync_copy(obs[0].at[pl.ds(0, 16)],
                            out_hbm.at[pl.ds(b * 16, 16)])


def _worker_body(idT_hbm, valT_hbm, emb_hbm, lw_hbm, out_hbm,
                 idx_v, idx2_v, val_v, ebuf, wbuf, obuf, sem0, sem1):
    sems = (sem0, sem1)
    nc = 2
    wid = lax.axis_index("s") * nc + lax.axis_index("c")
    base = wid * RPW

    pltpu.sync_copy(idT_hbm.at[:, pl.ds(base, RPW)], idx_v)
    pltpu.sync_copy(valT_hbm.at[:, pl.ds(base, RPW)], val_v)

    lanes = _lanes()

    ln2 = lanes * 2

    def dbl_body(j, _):
        for f in range(F):
            v2 = idx_v[f, pl.ds(j * 16, 16)] * 2
            plsc.store_scatter(idx2_v.at[f], [ln2 + j * 32], v2)
            plsc.store_scatter(idx2_v.at[f], [ln2 + (j * 32 + 1)], v2 + 1)
        return 0

    lax.fori_loop(0, RPW // 16, dbl_body, 0, unroll=False)

    def fire(c, par):
        waits = []
        for f in range(F):
            isl2 = idx2_v.at[f, pl.ds(c * CHR * 2, CHR * 2)]
            waits.append(pltpu.make_async_copy(
                emb_hbm.at[isl2], ebuf.at[par, f], sems[par]))
            isl = idx_v.at[f, pl.ds(c * CHR, CHR)]
            waits.append(pltpu.make_async_copy(
                lw_hbm.at[isl], wbuf.at[par, f], sems[par]))
        return waits

    def compute_chunk(c, par):
        def group_body(g, _):
            lb = g * 16
            gb = c * CHR + lb
            vseg = [val_v[f, pl.ds(gb, 16)] for f in range(F)]
            lin = jnp.zeros((16,), jnp.float32)
            for f in range(F):
                lin = lin + wbuf[par, f, pl.ds(lb, 16)] * vseg[f]
            ov = jnp.zeros((16,), jnp.float32)
            for r in range(16):
                acc = jnp.zeros((16,), jnp.float32)
                acc2 = jnp.zeros((16,), jnp.float32)
                rowi = jnp.full((16,), 2 * (lb + r), jnp.int32) + _idiv8()
                for f in range(F):
                    e = plsc.load_gather(
                        ebuf.at[par],
                        [jnp.full((16,), f, jnp.int32), rowi, _imod8()])
                    t = e * _bcast_lane(vseg[f], r)
                    acc = acc + t
                    acc2 = acc2 + t * t
                z = 0.5 * (acc * acc - acc2)
                ov = jnp.where(lanes == r, jnp.sum(z), ov)
            obuf[c * NG + g] = ov + lin
            return 0

        lax.fori_loop(0, NG, group_body, 0, unroll=False)

    for w in fire(0, 0):
        w.start()

    def cc_body(cc, _):
        for par in range(2):
            c = cc * 2 + par

            @pl.when(c + 1 < NCH)
            def _():
                for w in fire(c + 1, 1 - par):
                    w.start()

            for w in fire(c, par):
                w.wait()
            compute_chunk(c, par)
        return 0

    lax.fori_loop(0, NCH // 2, cc_body, 0, unroll=False)
    pltpu.sync_copy(obuf, out_hbm.at[pl.ds(wid * (RPW // 16), RPW // 16)])


@jax.jit
def _fm_sc(idT, valT, emb_table_t, lin_weight):
    mesh = plsc.VectorSubcoreMesh(core_axis_name="c", subcore_axis_name="s")

    relayout = pl.kernel(
        _relayout_body,
        out_type=jax.ShapeDtypeStruct((NFEAT // 8, 128), jnp.float32),
        mesh=mesh,
        compiler_params=pltpu.CompilerParams(
            needs_layout_passes=False, use_tc_tiling_on_sc=True),
        scratch_types=(
            [pltpu.VMEM((16, SB * 128), jnp.float32)] * 2
            + [pltpu.VMEM((SB * 16, 128), jnp.float32)] * 2
            + [pltpu.SemaphoreType.DMA] * 4
        ),
    )
    emb_lin = relayout(emb_table_t)
    tail = NBLK * 128
    patch = emb_table_t[:, tail:].T.reshape(NFEAT // 8 - NBLK * 16, 128)
    emb_lin = lax.dynamic_update_slice(emb_lin, patch, (NBLK * 16, 0))
    emb_lin = emb_lin.reshape(2 * NFEAT, 8)

    grid_kernel = pl.kernel(
        _worker_body,
        out_type=jax.ShapeDtypeStruct((B // 16, 16), jnp.float32),
        mesh=mesh,
        compiler_params=pltpu.CompilerParams(
            needs_layout_passes=False, use_tc_tiling_on_sc=False),
        scratch_types=[
            pltpu.VMEM((F, RPW), jnp.int32),
            pltpu.VMEM((F, 2 * RPW), jnp.int32),
            pltpu.VMEM((F, RPW), jnp.float32),
            pltpu.VMEM((2, F, 2 * CHR, 8), jnp.float32),
            pltpu.VMEM((2, F, CHR), jnp.float32),
            pltpu.VMEM((RPW // 16, 16), jnp.float32),
            pltpu.SemaphoreType.DMA,
            pltpu.SemaphoreType.DMA,
        ],
    )
    return grid_kernel(idT, valT, emb_lin, lin_weight)


def kernel(id, value, emb_table, lin_weight, lin_bias):
    out = _fm_sc(id.T, value.T, emb_table.T, lin_weight)
    return out.reshape(B) + lin_bias


# speedup vs baseline: 1.9534x; 1.0007x over previous
import jax
import jax.numpy as jnp
from jax import lax
from jax.experimental import pallas as pl
from jax.experimental.pallas import tpu as pltpu
from jax.experimental.pallas import tpu_sc as plsc

B, F, NFEAT, NEMB = 16384, 26, 1000000, 16
NW = 32
RPW = B // NW
CHR = 64
NCH = RPW // CHR
NG = CHR // 16

NBLK = NFEAT // 128
BPW = NBLK // NW
NREST = NBLK - BPW * NW


def _bcast_lane(v, j):
    idx = jnp.full((16, 1), j, jnp.int32)
    dnums = lax.GatherDimensionNumbers(
        offset_dims=(), collapsed_slice_dims=(0,), start_index_map=(0,))
    return lax.gather(v, idx, dnums, (1,),
                      mode=lax.GatherScatterMode.PROMISE_IN_BOUNDS)


def _lanes():
    return lax.iota(jnp.int32, 16)


def _idiv8():
    return lax.iota(jnp.int32, 16) // 8


def _imod8():
    return lax.iota(jnp.int32, 16) % 8


SB = 8
NSUP = BPW // SB
STAIL = BPW - NSUP * SB


def _transpose_dyn(tbuf, obuf_v, blk):
    ln = _lanes()
    cb = jnp.full((16,), blk * 128, jnp.int32)
    for rp in range(16):
        segs = [
            plsc.load_gather(
                tbuf, [ln, cb + jnp.full((16,), 8 * rp + s, jnp.int32)])
            for s in range(8)
        ]
        for s in range(8):
            obuf_v[blk * 16 + rp, pl.ds(16 * s, 16)] = segs[s]


def _relayout_body(embT_hbm, out_hbm, *bufs):
    nc = 2
    wid = lax.axis_index("s") * nc + lax.axis_index("c")
    tbs, obs = bufs[0:2], bufs[2:4]
    sis, sos = bufs[4:6], bufs[6:8]

    def in_copy(j, par, start):
        b = wid * BPW + j * SB
        for h in range(2):
            cp = pltpu.make_async_copy(
                embT_hbm.at[pl.ds(8 * h, 8), pl.ds(b * 128, SB * 128)],
                tbs[par].at[pl.ds(8 * h, 8)], sis[par])
            cp.start() if start else cp.wait()

    def out_copy(j, par, start):
        b = wid * BPW + j * SB
        cp = pltpu.make_async_copy(
            obs[par], out_hbm.at[pl.ds(b * 16, SB * 16)], sos[par])
        cp.start() if start else cp.wait()

    in_copy(0, 0, True)

    def jj_body(jj, _):
        for par in range(2):
            j = jj * 2 + par

            @pl.when(j + 1 < NSUP)
            def _():
                in_copy(j + 1, 1 - par, True)

            in_copy(j, par, False)

            @pl.when(j >= 2)
            def _():
                out_copy(j - 2, par, False)

            lax.fori_loop(
                0, SB,
                lambda blk, c: (_transpose_dyn(tbs[par], obs[par], blk), c)[1],
                0, unroll=False)
            out_copy(j, par, True)
        return 0

    lax.fori_loop(0, NSUP // 2, jj_body, 0, unroll=False)
    out_copy(NSUP - 2, 0, False)
    out_copy(NSUP - 1, 1, False)

    bt = wid * BPW + NSUP * SB
    for h in range(2):
        pltpu.sync_copy(
            embT_hbm.at[pl.ds(8 * h, 8), pl.ds(bt * 128, STAIL * 128)],
            tbs[0].at[pl.ds(8 * h, 8), pl.ds(0, STAIL * 128)])
    lax.fori_loop(
        0, STAIL,
        lambda blk, c: (_transpose_dyn(tbs[0], obs[0], blk), c)[1],
        0, unroll=False)
    pltpu.sync_copy(obs[0].at[pl.ds(0, STAIL * 16)],
                    out_hbm.at[pl.ds(bt * 16, STAIL * 16)])

    for w in range(NREST):
        @pl.when(wid == w)
        def _():
            b = NBLK - NREST + w
            for h in range(2):
                pltpu.sync_copy(
                    embT_hbm.at[pl.ds(8 * h, 8), pl.ds(b * 128, 128)],
                    tbs[0].at[pl.ds(8 * h, 8), pl.ds(0, 128)])
            _transpose_dyn(tbs[0], obs[0], 0)
            pltpu.sync_copy(obs[0].at[pl.ds(0, 16)],
                            out_hbm.at[pl.ds(b * 16, 16)])


def _worker_body(idT_hbm, valT_hbm, emb_hbm, lw_hbm, out_hbm,
                 idx_v, idx2_v, val_v, ebuf, wbuf, obuf, sem0, sem1):
    sems = (sem0, sem1)
    nc = 2
    wid = lax.axis_index("s") * nc + lax.axis_index("c")
    base = wid * RPW

    pltpu.sync_copy(idT_hbm.at[:, pl.ds(base, RPW)], idx_v)
    pltpu.sync_copy(valT_hbm.at[:, pl.ds(base, RPW)], val_v)

    lanes = _lanes()

    ln2 = lanes * 2

    def dbl_body(j, _):
        for f in range(F):
            v2 = idx_v[f, pl.ds(j * 16, 16)] * 2
            plsc.store_scatter(idx2_v.at[f], [ln2 + j * 32], v2)
            plsc.store_scatter(idx2_v.at[f], [ln2 + (j * 32 + 1)], v2 + 1)
        return 0

    lax.fori_loop(0, RPW // 16, dbl_body, 0, unroll=False)

    def fire(c, par):
        waits = []
        for f in range(F):
            isl2 = idx2_v.at[f, pl.ds(c * CHR * 2, CHR * 2)]
            waits.append(pltpu.make_async_copy(
                emb_hbm.at[isl2], ebuf.at[par, f], sems[par]))
            isl = idx_v.at[f, pl.ds(c * CHR, CHR)]
            waits.append(pltpu.make_async_copy(
                lw_hbm.at[isl], wbuf.at[par, f], sems[par]))
        return waits

    def compute_chunk(c, par):
        def group_body(g, _):
            lb = g * 16
            gb = c * CHR + lb
            vseg = [val_v[f, pl.ds(gb, 16)] for f in range(F)]
            lin = jnp.zeros((16,), jnp.float32)
            for f in range(F):
                lin = lin + wbuf[par, f, pl.ds(lb, 16)] * vseg[f]
            ov = jnp.zeros((16,), jnp.float32)
            for r in range(16):
                acc = jnp.zeros((16,), jnp.float32)
                acc2 = jnp.zeros((16,), jnp.float32)
                rowi = jnp.full((16,), 2 * (lb + r), jnp.int32) + _idiv8()
                for f in range(F):
                    e = plsc.load_gather(
                        ebuf.at[par],
                        [jnp.full((16,), f, jnp.int32), rowi, _imod8()])
                    t = e * _bcast_lane(vseg[f], r)
                    acc = acc + t
                    acc2 = acc2 + t * t
                z = 0.5 * (acc * acc - acc2)
                ov = jnp.where(lanes == r, jnp.sum(z), ov)
            obuf[c * NG + g] = ov + lin
            return 0

        lax.fori_loop(0, NG, group_body, 0, unroll=False)

    for w in fire(0, 0):
        w.start()

    def cc_body(cc, _):
        for par in range(2):
            c = cc * 2 + par

            @pl.when(c + 1 < NCH)
            def _():
                for w in fire(c + 1, 1 - par):
                    w.start()

            for w in fire(c, par):
                w.wait()
            compute_chunk(c, par)
        return 0

    lax.fori_loop(0, NCH // 2, cc_body, 0, unroll=False)
    pltpu.sync_copy(obuf, out_hbm.at[pl.ds(wid * (RPW // 16), RPW // 16)])


@jax.jit
def _fm_sc(idT, valT, emb_table_t, lin_weight):
    mesh = plsc.VectorSubcoreMesh(core_axis_name="c", subcore_axis_name="s")

    relayout = pl.kernel(
        _relayout_body,
        out_type=jax.ShapeDtypeStruct((NFEAT // 8, 128), jnp.float32),
        mesh=mesh,
        compiler_params=pltpu.CompilerParams(
            needs_layout_passes=False, use_tc_tiling_on_sc=True),
        scratch_types=(
            [pltpu.VMEM((16, SB * 128), jnp.float32)] * 2
            + [pltpu.VMEM((SB * 16, 128), jnp.float32)] * 2
            + [pltpu.SemaphoreType.DMA] * 4
        ),
    )
    emb_lin = relayout(emb_table_t)
    tail = NBLK * 128
    patch = emb_table_t[:, tail:].T.reshape(NFEAT // 8 - NBLK * 16, 128)
    emb_lin = lax.dynamic_update_slice(emb_lin, patch, (NBLK * 16, 0))
    emb_lin = emb_lin.reshape(2 * NFEAT, 8)

    grid_kernel = pl.kernel(
        _worker_body,
        out_type=jax.ShapeDtypeStruct((B // 16, 16), jnp.float32),
        mesh=mesh,
        compiler_params=pltpu.CompilerParams(
            needs_layout_passes=False, use_tc_tiling_on_sc=False),
        scratch_types=[
            pltpu.VMEM((F, RPW), jnp.int32),
            pltpu.VMEM((F, 2 * RPW), jnp.int32),
            pltpu.VMEM((F, RPW), jnp.float32),
            pltpu.VMEM((2, F, 2 * CHR, 8), jnp.float32),
            pltpu.VMEM((2, F, CHR), jnp.float32),
            pltpu.VMEM((RPW // 16, 16), jnp.float32),
            pltpu.SemaphoreType.DMA,
            pltpu.SemaphoreType.DMA,
        ],
    )
    return grid_kernel(idT, valT, emb_lin, lin_weight)


def kernel(id, value, emb_table, lin_weight, lin_bias):
    out = _fm_sc(id.T, value.T, emb_table.T, lin_weight)
    return out.reshape(B) + lin_bias
